# baseline (speedup 1.0000x reference)
import functools

import jax
import jax.numpy as jnp
from jax import lax
from jax.experimental import pallas as pl
from jax.experimental.pallas import tpu as pltpu
from jax.experimental.pallas import tpu_sc as plsc

W = 512
H = 512
B = 20000
NC = 2
NS = 16
NW = NC * NS
ROWS_PER_W = W // NW
BAND = ROWS_PER_W * H
GROUPS = B // 16


def _body(flat_hbm, out_hbm, boxes_v, c0_v, c1_v, c2_v):
    wid = lax.axis_index("s") * NC + lax.axis_index("c")
    row_lo = wid * ROWS_PER_W

    pltpu.sync_copy(flat_hbm, boxes_v)

    def _zero(i, _):
        z = jnp.zeros((16,), jnp.float32)
        c0_v[pl.ds(i * 16, 16)] = z
        c1_v[pl.ds(i * 16, 16)] = z
        c2_v[pl.ds(i * 16, 16)] = z
        return _
    lax.fori_loop(0, BAND // 16, _zero, None, unroll=8)

    lane = lax.iota(jnp.int32, 16)
    ones = jnp.ones((16,), jnp.float32)

    def _scan(g, _):
        pos = g * 64 + lane * 4
        x = plsc.load_gather(boxes_v, [pos])
        y = plsc.load_gather(boxes_v, [pos + 1])
        wv = plsc.load_gather(boxes_v, [pos + 2])
        hv = plsc.load_gather(boxes_v, [pos + 3])
        x0 = (x * jnp.float32(W)).astype(jnp.int32)
        y0 = (y * jnp.float32(H)).astype(jnp.int32)
        m = (x0 >= row_lo) & (x0 < row_lo + ROWS_PER_W)
        li = (x0 - row_lo) * H + y0
        li = jnp.where(m, li, 0)
        plsc.store_scatter(c0_v, [li], ones, mask=m)
        plsc.store_scatter(c1_v, [li], wv, mask=m)
        plsc.store_scatter(c2_v, [li], hv, mask=m)
        return _
    lax.fori_loop(0, GROUPS, _scan, None, unroll=8)

    pltpu.sync_copy(c0_v, out_hbm.at[pl.ds(0 * W * H + wid * BAND, BAND)])
    pltpu.sync_copy(c1_v, out_hbm.at[pl.ds(1 * W * H + wid * BAND, BAND)])
    pltpu.sync_copy(c2_v, out_hbm.at[pl.ds(2 * W * H + wid * BAND, BAND)])


@jax.jit
def _heatmap(flat_boxes):
    mesh = plsc.VectorSubcoreMesh(core_axis_name="c", subcore_axis_name="s")
    run = functools.partial(
        pl.kernel,
        mesh=mesh,
        compiler_params=pltpu.CompilerParams(needs_layout_passes=False),
        out_type=jax.ShapeDtypeStruct((3 * W * H,), jnp.float32),
        scratch_types=[
            pltpu.VMEM((B * 4,), jnp.float32),
            pltpu.VMEM((BAND,), jnp.float32),
            pltpu.VMEM((BAND,), jnp.float32),
            pltpu.VMEM((BAND,), jnp.float32),
        ],
    )(_body)
    return run(flat_boxes)


def kernel(boxes):
    flat = boxes.reshape(-1)
    return _heatmap(flat).reshape(1, 3, W, H)


# speedup vs baseline: 1.1542x; 1.1542x over previous
import functools

import jax
import jax.numpy as jnp
from jax import lax
from jax.experimental import pallas as pl
from jax.experimental.pallas import tpu as pltpu
from jax.experimental.pallas import tpu_sc as plsc

W = 512
H = 512
B = 20000
NC = 2
NS = 16
NW = NC * NS
ROWS_PER_W = W // NW
BAND = ROWS_PER_W * H
CHUNK_B = 2000
CHUNK_W = CHUNK_B * 4
NCHUNK = B // CHUNK_B
CHUNK_G = CHUNK_B // 16


def _body(flat_hbm, out_hbm, buf0_v, buf1_v, c0_v, c1_v, c2_v, sem0, sem1):
    wid = lax.axis_index("s") * NC + lax.axis_index("c")
    row_lo = wid * ROWS_PER_W

    bufs = (buf0_v, buf1_v)
    sems = (sem0, sem1)

    cp0 = pltpu.async_copy(flat_hbm.at[pl.ds(0, CHUNK_W)], buf0_v, sem0)

    def _zero(i, _):
        z = jnp.zeros((16,), jnp.float32)
        c0_v[pl.ds(i * 16, 16)] = z
        c1_v[pl.ds(i * 16, 16)] = z
        c2_v[pl.ds(i * 16, 16)] = z
        return _
    lax.fori_loop(0, BAND // 16, _zero, None, unroll=8)

    lane = lax.iota(jnp.int32, 16)
    ones = jnp.ones((16,), jnp.float32)

    def _scan(buf):
        def _g(g, _):
            pos = g * 64 + lane * 4
            x = plsc.load_gather(buf, [pos])
            y = plsc.load_gather(buf, [pos + 1])
            wv = plsc.load_gather(buf, [pos + 2])
            hv = plsc.load_gather(buf, [pos + 3])
            x0 = (x * jnp.float32(W)).astype(jnp.int32)
            y0 = (y * jnp.float32(H)).astype(jnp.int32)
            m = (x0 >= row_lo) & (x0 < row_lo + ROWS_PER_W)
            li = (x0 - row_lo) * H + y0
            li = jnp.where(m, li, 0)
            plsc.store_scatter(c0_v, [li], ones, mask=m)
            plsc.store_scatter(c1_v, [li], wv, mask=m)
            plsc.store_scatter(c2_v, [li], hv, mask=m)
            return _
        lax.fori_loop(0, CHUNK_G, _g, None)

    copies = [cp0]
    for k in range(NCHUNK):
        copies[k].wait()
        if k + 1 < NCHUNK:
            copies.append(pltpu.async_copy(
                flat_hbm.at[pl.ds((k + 1) * CHUNK_W, CHUNK_W)],
                bufs[(k + 1) % 2], sems[(k + 1) % 2]))
        _scan(bufs[k % 2])

    pltpu.sync_copy(c0_v, out_hbm.at[pl.ds(0 * W * H + wid * BAND, BAND)])
    pltpu.sync_copy(c1_v, out_hbm.at[pl.ds(1 * W * H + wid * BAND, BAND)])
    pltpu.sync_copy(c2_v, out_hbm.at[pl.ds(2 * W * H + wid * BAND, BAND)])


@jax.jit
def _heatmap(flat_boxes):
    mesh = plsc.VectorSubcoreMesh(core_axis_name="c", subcore_axis_name="s")
    run = functools.partial(
        pl.kernel,
        mesh=mesh,
        compiler_params=pltpu.CompilerParams(needs_layout_passes=False),
        out_type=jax.ShapeDtypeStruct((3 * W * H,), jnp.float32),
        scratch_types=[
            pltpu.VMEM((CHUNK_W,), jnp.float32),
            pltpu.VMEM((CHUNK_W,), jnp.float32),
            pltpu.VMEM((BAND,), jnp.float32),
            pltpu.VMEM((BAND,), jnp.float32),
            pltpu.VMEM((BAND,), jnp.float32),
            pltpu.SemaphoreType.DMA,
            pltpu.SemaphoreType.DMA,
        ],
    )(_body)
    return run(flat_boxes)


def kernel(boxes):
    flat = boxes.reshape(-1)
    return _heatmap(flat).reshape(1, 3, W, H)


# speedup vs baseline: 1.1931x; 1.0336x over previous
import functools

import jax
import jax.numpy as jnp
from jax import lax
from jax.experimental import pallas as pl
from jax.experimental.pallas import tpu as pltpu
from jax.experimental.pallas import tpu_sc as plsc

W = 512
H = 512
B = 20000
NC = 2
NS = 16
NW = NC * NS
ROWS_PER_W = W // NW
BAND = ROWS_PER_W * H
CHUNK_B = 2000
CHUNK_W = CHUNK_B * 4
NCHUNK = B // CHUNK_B
CHUNK_G = CHUNK_B // 16


def _body(flat_hbm, out_hbm, buf0_v, buf1_v, c0_v, c1_v, c2_v,
          pos_v, sem0, sem1):
    wid = lax.axis_index("s") * NC + lax.axis_index("c")
    row_lo = wid * ROWS_PER_W
    x_lo = wid.astype(jnp.float32) * jnp.float32(1.0 / NW)
    x_hi = (wid + 1).astype(jnp.float32) * jnp.float32(1.0 / NW)

    bufs = (buf0_v, buf1_v)
    sems = (sem0, sem1)

    cp0 = pltpu.async_copy(flat_hbm.at[pl.ds(0, CHUNK_W)], buf0_v, sem0)

    def _zero(i, _):
        z = jnp.zeros((16,), jnp.float32)
        c0_v[pl.ds(i * 16, 16)] = z
        c1_v[pl.ds(i * 16, 16)] = z
        c2_v[pl.ds(i * 16, 16)] = z
        return _
    lax.fori_loop(0, BAND // 16, _zero, None, unroll=8)

    lane = lax.iota(jnp.int32, 16)
    ones = jnp.ones((16,), jnp.float32)
    lane4 = lane * 4

    def _chunk(buf):
        def _p1(g, cnt):
            pos = g * 64 + lane4
            x = plsc.load_gather(buf, [pos])
            m = (x >= x_lo) & (x < x_hi)
            plsc.store_compressed(pos_v.at[pl.ds(cnt, 16)], pos, mask=m)
            return cnt + plsc.all_reduce_population_count(m)[0]
        cnt = lax.fori_loop(0, CHUNK_G, _p1, jnp.int32(0), unroll=4)

        def _p2(g, _):
            base = g * 16
            posv = pos_v[pl.ds(base, 16)]
            m2 = (base + lane) < cnt
            posv = jnp.where(m2, posv, 0)
            x = plsc.load_gather(buf, [posv])
            y = plsc.load_gather(buf, [posv + 1])
            wv = plsc.load_gather(buf, [posv + 2])
            hv = plsc.load_gather(buf, [posv + 3])
            x0 = (x * jnp.float32(W)).astype(jnp.int32)
            y0 = (y * jnp.float32(H)).astype(jnp.int32)
            li = (x0 - row_lo) * H + y0
            li = jnp.where(m2, li, 0)
            plsc.store_scatter(c0_v, [li], ones, mask=m2)
            plsc.store_scatter(c1_v, [li], wv, mask=m2)
            plsc.store_scatter(c2_v, [li], hv, mask=m2)
            return _
        lax.fori_loop(0, (cnt + 15) >> 4, _p2, None)

    copies = [cp0]
    for k in range(NCHUNK):
        copies[k].wait()
        if k + 1 < NCHUNK:
            copies.append(pltpu.async_copy(
                flat_hbm.at[pl.ds((k + 1) * CHUNK_W, CHUNK_W)],
                bufs[(k + 1) % 2], sems[(k + 1) % 2]))
        _chunk(bufs[k % 2])

    pltpu.sync_copy(c0_v, out_hbm.at[pl.ds(0 * W * H + wid * BAND, BAND)])
    pltpu.sync_copy(c1_v, out_hbm.at[pl.ds(1 * W * H + wid * BAND, BAND)])
    pltpu.sync_copy(c2_v, out_hbm.at[pl.ds(2 * W * H + wid * BAND, BAND)])


@jax.jit
def _heatmap(flat_boxes):
    mesh = plsc.VectorSubcoreMesh(core_axis_name="c", subcore_axis_name="s")
    run = functools.partial(
        pl.kernel,
        mesh=mesh,
        compiler_params=pltpu.CompilerParams(needs_layout_passes=False),
        out_type=jax.ShapeDtypeStruct((3 * W * H,), jnp.float32),
        scratch_types=[
            pltpu.VMEM((CHUNK_W,), jnp.float32),
            pltpu.VMEM((CHUNK_W,), jnp.float32),
            pltpu.VMEM((BAND,), jnp.float32),
            pltpu.VMEM((BAND,), jnp.float32),
            pltpu.VMEM((BAND,), jnp.float32),
            pltpu.VMEM((CHUNK_B + 16,), jnp.int32),
            pltpu.SemaphoreType.DMA,
            pltpu.SemaphoreType.DMA,
        ],
    )(_body)
    return run(flat_boxes)


def kernel(boxes):
    flat = boxes.reshape(-1)
    return _heatmap(flat).reshape(1, 3, W, H)


# speedup vs baseline: 1.4607x; 1.2243x over previous
import functools

import jax
import jax.numpy as jnp
from jax import lax
from jax.experimental import pallas as pl
from jax.experimental.pallas import tpu as pltpu
from jax.experimental.pallas import tpu_sc as plsc

W = 512
H = 512
B = 20000
NC = 2
NS = 16
NW = NC * NS
ROWS_PER_W = W // NW
BAND = ROWS_PER_W * H
GROUPS = B // 16


def _body(cols_hbm, out_hbm, x_v, y_v, w_v, h_v, ids_v, c0_v, c1_v, c2_v,
          semx, semy, semw, semh):
    wid = lax.axis_index("s") * NC + lax.axis_index("c")
    row_lo = wid * ROWS_PER_W
    x_lo = wid.astype(jnp.float32) * jnp.float32(1.0 / NW)
    x_hi = (wid + 1).astype(jnp.float32) * jnp.float32(1.0 / NW)

    cpx = pltpu.async_copy(cols_hbm.at[pl.ds(0 * B, B)], x_v, semx)
    cpy = pltpu.async_copy(cols_hbm.at[pl.ds(1 * B, B)], y_v, semy)
    cpw = pltpu.async_copy(cols_hbm.at[pl.ds(2 * B, B)], w_v, semw)
    cph = pltpu.async_copy(cols_hbm.at[pl.ds(3 * B, B)], h_v, semh)

    def _zero(i, _):
        z = jnp.zeros((16,), jnp.float32)
        c0_v[pl.ds(i * 16, 16)] = z
        c1_v[pl.ds(i * 16, 16)] = z
        c2_v[pl.ds(i * 16, 16)] = z
        return _
    lax.fori_loop(0, BAND // 16, _zero, None, unroll=8)

    lane = lax.iota(jnp.int32, 16)
    ones = jnp.ones((16,), jnp.float32)

    cpx.wait()

    def _p1(g, cnt):
        xv = x_v[pl.ds(g * 16, 16)]
        m = (xv >= x_lo) & (xv < x_hi)
        plsc.store_compressed(ids_v.at[pl.ds(cnt, 16)], g * 16 + lane, mask=m)
        return cnt + plsc.all_reduce_population_count(m)[0]
    cnt = lax.fori_loop(0, GROUPS, _p1, jnp.int32(0), unroll=4)

    cpy.wait()
    cpw.wait()
    cph.wait()

    def _p2(g, _):
        base = g * 16
        idv = ids_v[pl.ds(base, 16)]
        m2 = (base + lane) < cnt
        idv = jnp.where(m2, idv, 0)
        x = plsc.load_gather(x_v, [idv])
        y = plsc.load_gather(y_v, [idv])
        wv = plsc.load_gather(w_v, [idv])
        hv = plsc.load_gather(h_v, [idv])
        x0 = (x * jnp.float32(W)).astype(jnp.int32)
        y0 = (y * jnp.float32(H)).astype(jnp.int32)
        li = (x0 - row_lo) * H + y0
        li = jnp.where(m2, li, 0)
        plsc.store_scatter(c0_v, [li], ones, mask=m2)
        plsc.store_scatter(c1_v, [li], wv, mask=m2)
        plsc.store_scatter(c2_v, [li], hv, mask=m2)
        return _
    lax.fori_loop(0, (cnt + 15) >> 4, _p2, None)

    pltpu.sync_copy(c0_v, out_hbm.at[pl.ds(0 * W * H + wid * BAND, BAND)])
    pltpu.sync_copy(c1_v, out_hbm.at[pl.ds(1 * W * H + wid * BAND, BAND)])
    pltpu.sync_copy(c2_v, out_hbm.at[pl.ds(2 * W * H + wid * BAND, BAND)])


@jax.jit
def _heatmap(cols):
    mesh = plsc.VectorSubcoreMesh(core_axis_name="c", subcore_axis_name="s")
    run = functools.partial(
        pl.kernel,
        mesh=mesh,
        compiler_params=pltpu.CompilerParams(needs_layout_passes=False),
        out_type=jax.ShapeDtypeStruct((3 * W * H,), jnp.float32),
        scratch_types=[
            pltpu.VMEM((B,), jnp.float32),
            pltpu.VMEM((B,), jnp.float32),
            pltpu.VMEM((B,), jnp.float32),
            pltpu.VMEM((B,), jnp.float32),
            pltpu.VMEM((B + 16,), jnp.int32),
            pltpu.VMEM((BAND,), jnp.float32),
            pltpu.VMEM((BAND,), jnp.float32),
            pltpu.VMEM((BAND,), jnp.float32),
            pltpu.SemaphoreType.DMA,
            pltpu.SemaphoreType.DMA,
            pltpu.SemaphoreType.DMA,
            pltpu.SemaphoreType.DMA,
        ],
    )(_body)
    return run(cols)


def kernel(boxes):
    cols = boxes.T.reshape(-1)
    return _heatmap(cols).reshape(1, 3, W, H)
